# baseline (speedup 1.0000x reference)
import functools

import jax
import jax.numpy as jnp
from jax import lax
from jax.experimental import pallas as pl
from jax.experimental.pallas import tpu as pltpu
from jax.experimental.pallas import tpu_sc as plsc

N_FACTORS = 64
BATCH = 16384

_info = plsc.get_sparse_core_info()
_NC, _NS, _L = _info.num_cores, _info.num_subcores, _info.num_lanes
_NW = _NC * _NS
_BPW = BATCH // _NW


def _mf_body(user_hbm, movie_hbm, uf_hbm, mf_hbm, ub_hbm, mb_hbm, out_hbm,
             uidx_v, midx_v, urows_v, mrows_v, ub_v, mb_v, out_v, pacc_v,
             sem_u, sem_m, sem_ub, sem_mb):
    wid = lax.axis_index("s") * _NC + lax.axis_index("c")
    base = wid * _BPW

    pltpu.sync_copy(user_hbm.at[pl.ds(base, _BPW)], uidx_v)
    pltpu.sync_copy(movie_hbm.at[pl.ds(base, _BPW)], midx_v)

    cu = pltpu.make_async_copy(uf_hbm.at[uidx_v], urows_v, sem_u)
    cm = pltpu.make_async_copy(mf_hbm.at[midx_v], mrows_v, sem_m)
    cub = pltpu.make_async_copy(ub_hbm.at[uidx_v], ub_v, sem_ub)
    cmb = pltpu.make_async_copy(mb_hbm.at[midx_v], mb_v, sem_mb)
    cu.start(); cm.start(); cub.start(); cmb.start()
    cu.wait(); cm.wait(); cub.wait(); cmb.wait()

    lanes = lax.iota(jnp.int32, _L)

    def block(b, _):
        r0 = b * _L
        for r in range(_L):
            acc = urows_v[r0 + r, pl.ds(0, _L)] * mrows_v[r0 + r, pl.ds(0, _L)]
            for j in range(1, N_FACTORS // _L):
                acc = acc + (urows_v[r0 + r, pl.ds(j * _L, _L)]
                             * mrows_v[r0 + r, pl.ds(j * _L, _L)])
            pacc_v[pl.ds(r * _L, _L)] = acc
        tot = plsc.load_gather(pacc_v, [lanes * _L])
        for l in range(1, _L):
            tot = tot + plsc.load_gather(pacc_v, [lanes * _L + l])
        sl = pl.ds(r0, _L)
        out_v[sl] = tot + ub_v[sl] + mb_v[sl]
        return 0

    lax.fori_loop(0, _BPW // _L, block, 0)
    pltpu.sync_copy(out_v, out_hbm.at[pl.ds(base, _BPW)])


@jax.jit
def kernel(user, movie, user_factors, movie_factors, user_biases, movie_biases):
    mesh = plsc.VectorSubcoreMesh(core_axis_name="c", subcore_axis_name="s")
    run = pl.kernel(
        _mf_body,
        out_type=jax.ShapeDtypeStruct((BATCH,), jnp.float32),
        mesh=mesh,
        compiler_params=pltpu.CompilerParams(
            needs_layout_passes=False, use_tc_tiling_on_sc=False),
        scratch_types=[
            pltpu.VMEM((_BPW,), jnp.int32),
            pltpu.VMEM((_BPW,), jnp.int32),
            pltpu.VMEM((_BPW, N_FACTORS), jnp.float32),
            pltpu.VMEM((_BPW, N_FACTORS), jnp.float32),
            pltpu.VMEM((_BPW,), jnp.float32),
            pltpu.VMEM((_BPW,), jnp.float32),
            pltpu.VMEM((_BPW,), jnp.float32),
            pltpu.VMEM((_L * _L,), jnp.float32),
            pltpu.SemaphoreType.DMA,
            pltpu.SemaphoreType.DMA,
            pltpu.SemaphoreType.DMA,
            pltpu.SemaphoreType.DMA,
        ],
    )
    return run(user, movie, user_factors, movie_factors,
               user_biases.reshape(-1), movie_biases.reshape(-1))


# speedup vs baseline: 3.8077x; 3.8077x over previous
import functools

import jax
import jax.numpy as jnp
from jax import lax
from jax.experimental import pallas as pl
from jax.experimental.pallas import tpu as pltpu
from jax.experimental.pallas import tpu_sc as plsc

N_FACTORS = 64
BATCH = 16384
N_ROWS = 1000000

_info = plsc.get_sparse_core_info()
_NC, _NS, _L = _info.num_cores, _info.num_subcores, _info.num_lanes
_NW = _NC * _NS
_BPW = BATCH // _NW
_RING = 4
_NBLK = (N_ROWS + 127) // 128
_LASTW = N_ROWS - (_NBLK - 1) * 128


def _extract_body(blk_hbm, col_hbm, ft_hbm, out_hbm,
                  blk_v, col_v, rowbuf_v, slabs, tail_v, sem):
    wid = lax.axis_index("s") * _NC + lax.axis_index("c")
    base = wid * _BPW
    lanes = lax.iota(jnp.int32, _L)
    zeros = lanes * 0

    pltpu.sync_copy(blk_hbm.at[pl.ds(base, _BPW)], blk_v)
    pltpu.sync_copy(col_hbm.at[pl.ds(base, _BPW)], col_v)

    b_lo = blk_v[pl.ds(0, _L)][0]
    b_hi = blk_v[pl.ds(_BPW - _L, _L)][_L - 1]

    def slab_copy(s, b):
        off = pl.multiple_of(b * 128, 128)
        full = pltpu.make_async_copy(
            ft_hbm.at[pl.ds(0, N_FACTORS), pl.ds(off, 128)], slabs[s], sem)
        tailoff = (_NBLK - 1) * 128
        tail = pltpu.make_async_copy(
            ft_hbm.at[pl.ds(0, N_FACTORS), pl.ds(tailoff, _LASTW)],
            tail_v, sem)
        return full, tail

    def fire(s, b):
        full, tail = slab_copy(s, b)
        @pl.when(b < _NBLK - 1)
        def _():
            full.start()
        @pl.when(b == _NBLK - 1)
        def _():
            tail.start()

    def drain(s, b, b_hi):
        full, tail = slab_copy(s, b)
        @pl.when((b <= b_hi) & (b < _NBLK - 1))
        def _():
            full.wait()
        @pl.when((b <= b_hi) & (b == _NBLK - 1))
        def _():
            tail.wait()

    for s in range(_RING):
        @pl.when(b_lo + s <= b_hi)
        def _(s=s):
            fire(s, b_lo + s)

    def blk_at(k):
        v = plsc.load_gather(blk_v, [zeros + jnp.minimum(k, _BPW - 1)])[0]
        return jnp.where(k < _BPW, v, jnp.int32(-1))

    def ring_step(j, carry):
        k = carry
        for s in range(_RING):
            b = b_lo + j * _RING + s
            drain(s, b, b_hi)

            def ext_cond(k2):
                return blk_at(k2) == b

            def ext_body(k2):
                c = plsc.load_gather(col_v, [zeros + k2])[0]
                ct = jnp.minimum(c, _LASTW - 1)
                is_tail = b == _NBLK - 1
                for q in range(N_FACTORS // _L):
                    vec = plsc.load_gather(slabs[s], [q * _L + lanes, zeros + c])
                    tvec = plsc.load_gather(tail_v, [q * _L + lanes, zeros + ct])
                    rowbuf_v[k2, pl.ds(q * _L, _L)] = jnp.where(is_tail, tvec, vec)
                return k2 + 1

            k = lax.while_loop(ext_cond, ext_body, k)
            bn = b + _RING
            @pl.when(bn <= b_hi)
            def _(s=s, bn=bn):
                fire(s, bn)
        return k

    nsteps = (b_hi - b_lo + _RING) // _RING
    lax.fori_loop(0, nsteps, ring_step, jnp.int32(0))
    pltpu.sync_copy(rowbuf_v, out_hbm.at[pl.ds(base, _BPW)])


def _gather_a_body(ublk_hbm, ucol_hbm, mblk_hbm, mcol_hbm, uft_hbm, mft_hbm,
                   urows_hbm, mrows_hbm,
                   blk_v, col_v, rowbuf_v, s0, s1, s2, s3, tail_v, sem):
    slabs = (s0, s1, s2, s3)
    _extract_body(ublk_hbm, ucol_hbm, uft_hbm, urows_hbm,
                  blk_v, col_v, rowbuf_v, slabs, tail_v, sem)
    _extract_body(mblk_hbm, mcol_hbm, mft_hbm, mrows_hbm,
                  blk_v, col_v, rowbuf_v, slabs, tail_v, sem)


def _gather_b_body(ipu_hbm, ipm_hbm, user_hbm, movie_hbm,
                   urows_s_hbm, mrows_s_hbm, ub_hbm, mb_hbm, out_hbm,
                   ipu_v, ipm_v, uidx_v, midx_v,
                   urows_v, mrows_v, ub_v, mb_v, out_v, pacc_v,
                   sem_u, sem_m, sem_ub, sem_mb):
    wid = lax.axis_index("s") * _NC + lax.axis_index("c")
    base = wid * _BPW

    pltpu.sync_copy(ipu_hbm.at[pl.ds(base, _BPW)], ipu_v)
    pltpu.sync_copy(ipm_hbm.at[pl.ds(base, _BPW)], ipm_v)
    pltpu.sync_copy(user_hbm.at[pl.ds(base, _BPW)], uidx_v)
    pltpu.sync_copy(movie_hbm.at[pl.ds(base, _BPW)], midx_v)

    cu = pltpu.make_async_copy(urows_s_hbm.at[ipu_v], urows_v, sem_u)
    cm = pltpu.make_async_copy(mrows_s_hbm.at[ipm_v], mrows_v, sem_m)
    cub = pltpu.make_async_copy(ub_hbm.at[uidx_v], ub_v, sem_ub)
    cmb = pltpu.make_async_copy(mb_hbm.at[midx_v], mb_v, sem_mb)
    cu.start(); cm.start(); cub.start(); cmb.start()
    cu.wait(); cm.wait(); cub.wait(); cmb.wait()

    lanes = lax.iota(jnp.int32, _L)

    def block(b, _):
        r0 = b * _L
        for r in range(_L):
            acc = urows_v[r0 + r, pl.ds(0, _L)] * mrows_v[r0 + r, pl.ds(0, _L)]
            for j in range(1, N_FACTORS // _L):
                acc = acc + (urows_v[r0 + r, pl.ds(j * _L, _L)]
                             * mrows_v[r0 + r, pl.ds(j * _L, _L)])
            pacc_v[pl.ds(r * _L, _L)] = acc
        tot = plsc.load_gather(pacc_v, [lanes * _L])
        for l in range(1, _L):
            tot = tot + plsc.load_gather(pacc_v, [lanes * _L + l])
        sl = pl.ds(r0, _L)
        out_v[sl] = tot + ub_v[sl] + mb_v[sl]
        return 0

    lax.fori_loop(0, _BPW // _L, block, 0)
    pltpu.sync_copy(out_v, out_hbm.at[pl.ds(base, _BPW)])


@jax.jit
def kernel(user, movie, user_factors, movie_factors, user_biases, movie_biases):
    mesh = plsc.VectorSubcoreMesh(core_axis_name="c", subcore_axis_name="s")

    run_a = pl.kernel(
        _gather_a_body,
        out_type=(jax.ShapeDtypeStruct((BATCH, N_FACTORS), jnp.float32),
                  jax.ShapeDtypeStruct((BATCH, N_FACTORS), jnp.float32)),
        mesh=mesh,
        compiler_params=pltpu.CompilerParams(
            needs_layout_passes=False, use_tc_tiling_on_sc=True),
        scratch_types=[
            pltpu.VMEM((_BPW,), jnp.int32),
            pltpu.VMEM((_BPW,), jnp.int32),
            pltpu.VMEM((_BPW, N_FACTORS), jnp.float32),
            pltpu.VMEM((N_FACTORS, 128), jnp.float32),
            pltpu.VMEM((N_FACTORS, 128), jnp.float32),
            pltpu.VMEM((N_FACTORS, 128), jnp.float32),
            pltpu.VMEM((N_FACTORS, 128), jnp.float32),
            pltpu.VMEM((N_FACTORS, _LASTW), jnp.float32),
            pltpu.SemaphoreType.DMA,
        ],
    )

    run_b = pl.kernel(
        _gather_b_body,
        out_type=jax.ShapeDtypeStruct((BATCH,), jnp.float32),
        mesh=mesh,
        compiler_params=pltpu.CompilerParams(
            needs_layout_passes=False, use_tc_tiling_on_sc=False),
        scratch_types=[
            pltpu.VMEM((_BPW,), jnp.int32),
            pltpu.VMEM((_BPW,), jnp.int32),
            pltpu.VMEM((_BPW,), jnp.int32),
            pltpu.VMEM((_BPW,), jnp.int32),
            pltpu.VMEM((_BPW, N_FACTORS), jnp.float32),
            pltpu.VMEM((_BPW, N_FACTORS), jnp.float32),
            pltpu.VMEM((_BPW,), jnp.float32),
            pltpu.VMEM((_BPW,), jnp.float32),
            pltpu.VMEM((_BPW,), jnp.float32),
            pltpu.VMEM((_L * _L,), jnp.float32),
            pltpu.SemaphoreType.DMA,
            pltpu.SemaphoreType.DMA,
            pltpu.SemaphoreType.DMA,
            pltpu.SemaphoreType.DMA,
        ],
    )

    iot = jnp.arange(BATCH, dtype=jnp.int32)
    pu = jnp.argsort(user).astype(jnp.int32)
    pm = jnp.argsort(movie).astype(jnp.int32)
    su = jnp.take(user, pu)
    sm = jnp.take(movie, pm)
    inv_pu = jnp.zeros((BATCH,), jnp.int32).at[pu].set(iot)
    inv_pm = jnp.zeros((BATCH,), jnp.int32).at[pm].set(iot)

    urows_s, mrows_s = run_a(su >> 7, su & 127, sm >> 7, sm & 127,
                             user_factors.T, movie_factors.T)
    return run_b(inv_pu, inv_pm, user, movie, urows_s, mrows_s,
                 user_biases.reshape(-1), movie_biases.reshape(-1))
